# baseline (speedup 1.0000x reference)
import functools
import math

import jax
import jax.numpy as jnp
from jax import lax
from jax.experimental import pallas as pl
from jax.experimental.pallas import tpu as pltpu
from jax.experimental.pallas import tpu_sc as plsc


def _proj_body(xq, xk, xv, wq, wk, wv, bq, bk, bv, qo, ko, vo):
    qo[...] = jnp.dot(xq[...], wq[...], preferred_element_type=jnp.float32) + bq[...]
    ko[...] = jnp.dot(xk[...], wk[...], preferred_element_type=jnp.float32) + bk[...]
    vo[...] = jnp.dot(xv[...], wv[...], preferred_element_type=jnp.float32) + bv[...]


def _projections(xq, xk, xv, Wq, bq, Wk, bk, Wv, bv):
    M, D = xq.shape
    TM = 512
    grid = (M // TM,)
    row = lambda i: (i, 0)
    fixed = lambda i: (0, 0)
    out = pl.pallas_call(
        _proj_body,
        grid=grid,
        in_specs=[
            pl.BlockSpec((TM, D), row),
            pl.BlockSpec((TM, D), row),
            pl.BlockSpec((TM, D), row),
            pl.BlockSpec((D, D), fixed),
            pl.BlockSpec((D, D), fixed),
            pl.BlockSpec((D, D), fixed),
            pl.BlockSpec((1, D), fixed),
            pl.BlockSpec((1, D), fixed),
            pl.BlockSpec((1, D), fixed),
        ],
        out_specs=[
            pl.BlockSpec((TM, D), row),
            pl.BlockSpec((TM, D), row),
            pl.BlockSpec((TM, D), row),
        ],
        out_shape=[jax.ShapeDtypeStruct((M, D), jnp.float32)] * 3,
    )(xq, xk, xv, Wq, Wk, Wv, bq.reshape(1, D), bk.reshape(1, D), bv.reshape(1, D))
    return out


def _corr_body(q_ref, k_ref, mv_ref):
    L = q_ref.shape[1]
    TJ = k_ref.shape[1]
    j = pl.program_id(1)
    q = q_ref[0]
    k = k_ref[0]
    g = lax.dot_general(q, k, (((1,), (1,)), ((), ())),
                        preferred_element_type=jnp.float32)
    lane = lax.broadcasted_iota(jnp.int32, (L, TJ), 1)
    x = g
    nbits = int(math.log2(TJ))
    for kb in range(nbits):
        sh = 1 << kb
        rolled = jnp.roll(x, -sh, axis=0)
        x = jnp.where((lane & sh) != 0, rolled, x)
    contrib = jnp.sum(x, axis=1, keepdims=True)
    shifted = pltpu.roll(contrib, L - j * TJ, axis=0)
    scaled = shifted * (1.0 / q_ref.shape[2])

    @pl.when(j == 0)
    def _():
        mv_ref[0] = scaled

    @pl.when(j > 0)
    def _():
        mv_ref[0] += scaled


def _mean_corr(q3, k3):
    B, L, D = q3.shape
    TJ = 128
    grid = (B, L // TJ)
    mv = pl.pallas_call(
        _corr_body,
        grid=grid,
        in_specs=[
            pl.BlockSpec((1, L, D), lambda b, j: (b, 0, 0)),
            pl.BlockSpec((1, TJ, D), lambda b, j: (b, j, 0)),
        ],
        out_specs=pl.BlockSpec((1, L, 1), lambda b, j: (b, 0, 0)),
        out_shape=jax.ShapeDtypeStruct((B, L, 1), jnp.float32),
        compiler_params=pltpu.CompilerParams(
            dimension_semantics=("arbitrary", "arbitrary")),
    )(q3, k3)
    return mv.reshape(B, L)


def _topk_body(mv_ref, idx_ref, sw_ref, *, topk):
    mv = mv_ref[...]
    Bb, L = mv.shape
    bm = jnp.mean(mv, axis=0, keepdims=True)
    colL = lax.broadcasted_iota(jnp.int32, (1, L), 1)
    colS = lax.broadcasted_iota(jnp.int32, (1, 128), 1)
    idx_row = jnp.zeros((1, 128), jnp.int32)
    w_acc = jnp.zeros((Bb, 128), jnp.float32)
    for p in range(topk):
        mx = jnp.max(bm)
        cand = jnp.where(bm == mx, colL, jnp.int32(2**30))
        fidx = jnp.min(cand)
        idx_row = jnp.where(colS == p, fidx, idx_row)
        wcol = jnp.sum(jnp.where(colL == fidx, mv, 0.0), axis=1, keepdims=True)
        w_acc = jnp.where(colS == p, wcol, w_acc)
        bm = jnp.where(colL == fidx, -jnp.inf, bm)
    mask = colS < topk
    m = jnp.max(jnp.where(mask, w_acc, -jnp.inf), axis=1, keepdims=True)
    e = jnp.where(mask, jnp.exp(w_acc - m), 0.0)
    sw = e / jnp.sum(e, axis=1, keepdims=True)
    idx_ref[...] = idx_row
    sw_ref[...] = sw


def _topk_weights(mv, topk):
    B, L = mv.shape
    idx, sw = pl.pallas_call(
        functools.partial(_topk_body, topk=topk),
        grid=(1,),
        in_specs=[pl.BlockSpec((B, L), lambda i: (0, 0))],
        out_specs=[
            pl.BlockSpec((1, 128), lambda i: (0, 0)),
            pl.BlockSpec((B, 128), lambda i: (0, 0)),
        ],
        out_shape=[
            jax.ShapeDtypeStruct((1, 128), jnp.int32),
            jax.ShapeDtypeStruct((B, 128), jnp.float32),
        ],
    )(mv)
    return idx, sw


_GDN = lax.GatherDimensionNumbers(
    offset_dims=(), collapsed_slice_dims=(0,), start_index_map=(0,))


def _lane_shuffle(x, idx):
    return lax.gather(x, idx[:, None], _GDN, slice_sizes=(1,),
                      mode=lax.GatherScatterMode.PROMISE_IN_BOUNDS)


def _all_lanes(x, op, lanes):
    for d in (1, 2, 4, 8):
        x = op(x, _lane_shuffle(x, lanes ^ d))
    return x


def _topk_sc_body(mv_hbm, idx_hbm, sw_hbm, mv_v, bm_v, idx_v, sw_v, *, topk):
    Bb, L = mv_hbm.shape
    c = lax.axis_index("c")
    s = lax.axis_index("s")

    @pl.when(jnp.logical_and(c == 0, s == 0))
    def _():
        pltpu.sync_copy(mv_hbm, mv_v)
        lanes = lax.iota(jnp.int32, 16)

        def bmloop(i, carry):
            acc = mv_v[0, pl.ds(i * 16, 16)]
            for b in range(1, Bb):
                acc = acc + mv_v[b, pl.ds(i * 16, 16)]
            bm_v[pl.ds(i * 16, 16)] = acc * (1.0 / Bb)
            return carry

        lax.fori_loop(0, L // 16, bmloop, 0)

        idx_acc = jnp.zeros((16,), jnp.int32)
        w_acc = [jnp.zeros((16,), jnp.float32) for _ in range(Bb)]
        for p in range(topk):
            def scanmax(j, carry):
                m_val, m_idx = carry
                v = bm_v[pl.ds(j * 16, 16)]
                idxs = j * 16 + lanes
                upd = v > m_val
                return (jnp.where(upd, v, m_val), jnp.where(upd, idxs, m_idx))

            m_val, m_idx = lax.fori_loop(
                0, L // 16, scanmax,
                (jnp.full((16,), -jnp.inf, jnp.float32),
                 jnp.zeros((16,), jnp.int32)))
            mxv = _all_lanes(m_val, jnp.maximum, lanes)
            cand = jnp.where(m_val == mxv, m_idx, jnp.int32(2**30))
            fidx = _all_lanes(cand, jnp.minimum, lanes)
            idx_acc = jnp.where(lanes == p, fidx, idx_acc)
            fidx_s = fidx[0]
            base = (fidx_s // 16) * 16
            hit = base + lanes == fidx_s
            ch = bm_v[pl.ds(base, 16)]
            bm_v[pl.ds(base, 16)] = jnp.where(
                hit, jnp.full((16,), -jnp.inf, jnp.float32), ch)
            for b in range(Bb):
                row = mv_v[b, pl.ds(base, 16)]
                wv = _all_lanes(jnp.where(hit, row, 0.0), jnp.add, lanes)
                w_acc[b] = jnp.where(lanes == p, wv, w_acc[b])

        idx_v[...] = idx_acc
        msk = lanes < topk
        for b in range(Bb):
            wm = jnp.where(msk, w_acc[b], -jnp.inf)
            mxw = _all_lanes(wm, jnp.maximum, lanes)
            e = jnp.where(msk, jnp.exp(wm - mxw), 0.0)
            ssum = _all_lanes(e, jnp.add, lanes)
            sw_v[b, pl.ds(0, 16)] = e / ssum
        pltpu.sync_copy(idx_v, idx_hbm)
        pltpu.sync_copy(sw_v, sw_hbm)


def _topk_weights_sc(mv, topk):
    B, L = mv.shape
    mesh = plsc.VectorSubcoreMesh(core_axis_name="c", subcore_axis_name="s")
    f = pl.kernel(
        functools.partial(_topk_sc_body, topk=topk),
        out_type=[
            jax.ShapeDtypeStruct((16,), jnp.int32),
            jax.ShapeDtypeStruct((B, 16), jnp.float32),
        ],
        mesh=mesh,
        scratch_types=[
            pltpu.VMEM((B, L), jnp.float32),
            pltpu.VMEM((L,), jnp.float32),
            pltpu.VMEM((16,), jnp.int32),
            pltpu.VMEM((B, 16), jnp.float32),
        ],
    )
    idx, sw = f(mv)
    return idx.reshape(1, 16), sw


def _agg_body(idx_ref, sw_ref, v_ref, out_ref, *, topk):
    b = pl.program_id(0)
    v = v_ref[0]
    L = v.shape[0]
    out_ref[0] = sw_ref[b, 0] * pltpu.roll(v, L - idx_ref[0, 0], axis=0)
    for i in range(1, topk):
        out_ref[0] += sw_ref[b, i] * pltpu.roll(v, L - idx_ref[0, i], axis=0)


def _aggregate(v3, idx, sw, topk):
    B, L, D = v3.shape
    out = pl.pallas_call(
        functools.partial(_agg_body, topk=topk),
        grid=(B,),
        in_specs=[
            pl.BlockSpec(memory_space=pltpu.SMEM),
            pl.BlockSpec(memory_space=pltpu.SMEM),
            pl.BlockSpec((1, L, D), lambda b: (b, 0, 0)),
        ],
        out_specs=pl.BlockSpec((1, L, D), lambda b: (b, 0, 0)),
        out_shape=jax.ShapeDtypeStruct((B, L, D), jnp.float32),
    )(idx, sw, v3)
    return out


def _outproj_body(x_ref, w_ref, b_ref, o_ref):
    o_ref[...] = jnp.dot(x_ref[...], w_ref[...],
                         preferred_element_type=jnp.float32) + b_ref[...]


def _out_projection(x, Wo, bo):
    M, D = x.shape
    TM = 512
    return pl.pallas_call(
        _outproj_body,
        grid=(M // TM,),
        in_specs=[
            pl.BlockSpec((TM, D), lambda i: (i, 0)),
            pl.BlockSpec((D, D), lambda i: (0, 0)),
            pl.BlockSpec((1, D), lambda i: (0, 0)),
        ],
        out_specs=pl.BlockSpec((TM, D), lambda i: (i, 0)),
        out_shape=jax.ShapeDtypeStruct((M, D), jnp.float32),
    )(x, Wo, bo.reshape(1, D))


def kernel(queries, keys, values, Wq, bq, Wk, bk, Wv, bv, Wo, bo):
    B, L, D = queries.shape
    topk = int(math.log(L))
    q2, k2, v2 = _projections(
        queries.reshape(B * L, D), keys.reshape(B * L, D),
        values.reshape(B * L, D), Wq, bq, Wk, bk, Wv, bv)
    q3 = q2.reshape(B, L, D)
    k3 = k2.reshape(B, L, D)
    v3 = v2.reshape(B, L, D)
    mv = _mean_corr(q3, k3)
    idx, sw = _topk_weights_sc(mv, topk)
    agg = _aggregate(v3, idx, sw, topk)
    out = _out_projection(agg.reshape(B * L, D), Wo, bo)
    return out.reshape(B, L, D)


# speedup vs baseline: 1.1835x; 1.1835x over previous
import functools
import math

import jax
import jax.numpy as jnp
from jax import lax
from jax.experimental import pallas as pl
from jax.experimental.pallas import tpu as pltpu
from jax.experimental.pallas import tpu_sc as plsc


def _prep_body(wq, wk, wv, wo, bv, bo, m_ref, wvo_ref, bvo_ref):
    m_ref[...] = lax.dot_general(wq[...], wk[...], (((1,), (1,)), ((), ())),
                                 preferred_element_type=jnp.float32)
    wvo_ref[...] = jnp.dot(wv[...], wo[...], preferred_element_type=jnp.float32)
    bvo_ref[...] = jnp.dot(bv[...], wo[...],
                           preferred_element_type=jnp.float32) + bo[...]


def _prep(Wq, Wk, Wv, Wo, bv, bo):
    D = Wq.shape[0]
    fixed = lambda: (0, 0)
    return pl.pallas_call(
        _prep_body,
        in_specs=[pl.BlockSpec((D, D), fixed)] * 4
        + [pl.BlockSpec((1, D), fixed)] * 2,
        out_specs=[
            pl.BlockSpec((D, D), fixed),
            pl.BlockSpec((D, D), fixed),
            pl.BlockSpec((1, D), fixed),
        ],
        out_shape=[
            jax.ShapeDtypeStruct((D, D), jnp.float32),
            jax.ShapeDtypeStruct((D, D), jnp.float32),
            jax.ShapeDtypeStruct((1, D), jnp.float32),
        ],
    )(Wq, Wk, Wv, Wo, bv.reshape(1, D), bo.reshape(1, D))


def _proj_body(xq, xv, m, wvo, bvo, qm_out, vp_out):
    qm_out[...] = jnp.dot(xq[...], m[...], preferred_element_type=jnp.float32)
    vp_out[...] = jnp.dot(xv[...], wvo[...],
                          preferred_element_type=jnp.float32) + bvo[...]


def _projections(xq, xv, M, Wvo, bvo):
    Mrows, D = xq.shape
    TM = 512
    row = lambda i: (i, 0)
    fixed = lambda i: (0, 0)
    return pl.pallas_call(
        _proj_body,
        grid=(Mrows // TM,),
        in_specs=[
            pl.BlockSpec((TM, D), row),
            pl.BlockSpec((TM, D), row),
            pl.BlockSpec((D, D), fixed),
            pl.BlockSpec((D, D), fixed),
            pl.BlockSpec((1, D), fixed),
        ],
        out_specs=[
            pl.BlockSpec((TM, D), row),
            pl.BlockSpec((TM, D), row),
        ],
        out_shape=[jax.ShapeDtypeStruct((Mrows, D), jnp.float32)] * 2,
    )(xq, xv, M, Wvo, bvo)


def _corr_body(q_ref, k_ref, mv_ref):
    L = q_ref.shape[1]
    TJ = k_ref.shape[1]
    j = pl.program_id(1)
    q = q_ref[0]
    k = k_ref[0]
    g = lax.dot_general(q, k, (((1,), (1,)), ((), ())),
                        preferred_element_type=jnp.float32)
    lane = lax.broadcasted_iota(jnp.int32, (L, TJ), 1)
    x = g
    nbits = int(math.log2(TJ))
    for kb in range(nbits):
        sh = 1 << kb
        rolled = jnp.roll(x, -sh, axis=0)
        x = jnp.where((lane & sh) != 0, rolled, x)
    contrib = jnp.sum(x, axis=1, keepdims=True)
    shifted = pltpu.roll(contrib, L - j * TJ, axis=0)
    scaled = shifted * (1.0 / q_ref.shape[2])

    @pl.when(j == 0)
    def _():
        mv_ref[0] = scaled

    @pl.when(j > 0)
    def _():
        mv_ref[0] += scaled


def _mean_corr(q3, k3):
    B, L, D = q3.shape
    TJ = 256
    grid = (B, L // TJ)
    mv = pl.pallas_call(
        _corr_body,
        grid=grid,
        in_specs=[
            pl.BlockSpec((1, L, D), lambda b, j: (b, 0, 0)),
            pl.BlockSpec((1, TJ, D), lambda b, j: (b, j, 0)),
        ],
        out_specs=pl.BlockSpec((1, L, 1), lambda b, j: (b, 0, 0)),
        out_shape=jax.ShapeDtypeStruct((B, L, 1), jnp.float32),
        compiler_params=pltpu.CompilerParams(
            dimension_semantics=("arbitrary", "arbitrary")),
    )(q3, k3)
    return mv.reshape(B, L)


_GDN = lax.GatherDimensionNumbers(
    offset_dims=(), collapsed_slice_dims=(0,), start_index_map=(0,))


def _lane_shuffle(x, idx):
    return lax.gather(x, idx[:, None], _GDN, slice_sizes=(1,),
                      mode=lax.GatherScatterMode.PROMISE_IN_BOUNDS)


def _all_lanes(x, op, lanes):
    for d in (1, 2, 4, 8):
        x = op(x, _lane_shuffle(x, lanes ^ d))
    return x


def _topk_sc_body(mv_hbm, idx_hbm, sw_hbm, mv_v, bm_v, idx_v, sw_v, *, topk):
    Bb, L = mv_hbm.shape
    c = lax.axis_index("c")
    s = lax.axis_index("s")

    @pl.when(jnp.logical_and(c == 0, s == 0))
    def _():
        pltpu.sync_copy(mv_hbm, mv_v)
        lanes = lax.iota(jnp.int32, 16)

        def bmloop(i, carry):
            acc = mv_v[0, pl.ds(i * 16, 16)]
            for b in range(1, Bb):
                acc = acc + mv_v[b, pl.ds(i * 16, 16)]
            bm_v[pl.ds(i * 16, 16)] = acc * (1.0 / Bb)
            return carry

        lax.fori_loop(0, L // 16, bmloop, 0)

        idx_acc = jnp.zeros((16,), jnp.int32)
        w_acc = [jnp.zeros((16,), jnp.float32) for _ in range(Bb)]
        for p in range(topk):
            def scanmax(j, carry):
                m_val, m_idx = carry
                v = bm_v[pl.ds(j * 16, 16)]
                idxs = j * 16 + lanes
                upd = v > m_val
                return (jnp.where(upd, v, m_val), jnp.where(upd, idxs, m_idx))

            m_val, m_idx = lax.fori_loop(
                0, L // 16, scanmax,
                (jnp.full((16,), -jnp.inf, jnp.float32),
                 jnp.zeros((16,), jnp.int32)))
            mxv = _all_lanes(m_val, jnp.maximum, lanes)
            cand = jnp.where(m_val == mxv, m_idx, jnp.int32(2**30))
            fidx = _all_lanes(cand, jnp.minimum, lanes)
            idx_acc = jnp.where(lanes == p, fidx, idx_acc)
            fidx_s = fidx[0]
            base = (fidx_s // 16) * 16
            hit = base + lanes == fidx_s
            ch = bm_v[pl.ds(base, 16)]
            bm_v[pl.ds(base, 16)] = jnp.where(
                hit, jnp.full((16,), -jnp.inf, jnp.float32), ch)
            for b in range(Bb):
                row = mv_v[b, pl.ds(base, 16)]
                wv = _all_lanes(jnp.where(hit, row, 0.0), jnp.add, lanes)
                w_acc[b] = jnp.where(lanes == p, wv, w_acc[b])

        idx_v[...] = idx_acc
        msk = lanes < topk
        for b in range(Bb):
            wm = jnp.where(msk, w_acc[b], -jnp.inf)
            mxw = _all_lanes(wm, jnp.maximum, lanes)
            e = jnp.where(msk, jnp.exp(wm - mxw), 0.0)
            ssum = _all_lanes(e, jnp.add, lanes)
            sw_v[b, pl.ds(0, 16)] = e / ssum
        pltpu.sync_copy(idx_v, idx_hbm)
        pltpu.sync_copy(sw_v, sw_hbm)


def _topk_weights_sc(mv, topk):
    B, L = mv.shape
    mesh = plsc.VectorSubcoreMesh(core_axis_name="c", subcore_axis_name="s")
    f = pl.kernel(
        functools.partial(_topk_sc_body, topk=topk),
        out_type=[
            jax.ShapeDtypeStruct((16,), jnp.int32),
            jax.ShapeDtypeStruct((B, 16), jnp.float32),
        ],
        mesh=mesh,
        scratch_types=[
            pltpu.VMEM((B, L), jnp.float32),
            pltpu.VMEM((L,), jnp.float32),
            pltpu.VMEM((16,), jnp.int32),
            pltpu.VMEM((B, 16), jnp.float32),
        ],
    )
    idx, sw = f(mv)
    return idx.reshape(1, 16), sw


def _agg_body(idx_ref, sw_ref, v_ref, out_ref, *, topk):
    b = pl.program_id(0)
    v = v_ref[0]
    L = v.shape[0]
    out_ref[0] = sw_ref[b, 0] * pltpu.roll(v, L - idx_ref[0, 0], axis=0)
    for i in range(1, topk):
        out_ref[0] += sw_ref[b, i] * pltpu.roll(v, L - idx_ref[0, i], axis=0)


def _aggregate(v3, idx, sw, topk):
    B, L, D = v3.shape
    out = pl.pallas_call(
        functools.partial(_agg_body, topk=topk),
        grid=(B,),
        in_specs=[
            pl.BlockSpec(memory_space=pltpu.SMEM),
            pl.BlockSpec(memory_space=pltpu.SMEM),
            pl.BlockSpec((1, L, D), lambda b: (b, 0, 0)),
        ],
        out_specs=pl.BlockSpec((1, L, D), lambda b: (b, 0, 0)),
        out_shape=jax.ShapeDtypeStruct((B, L, D), jnp.float32),
    )(idx, sw, v3)
    return out


def kernel(queries, keys, values, Wq, bq, Wk, bk, Wv, bv, Wo, bo):
    B, L, D = queries.shape
    topk = int(math.log(L))
    M, Wvo, bvo = _prep(Wq, Wk, Wv, Wo, bv, bo)
    qm, vp = _projections(
        queries.reshape(B * L, D), values.reshape(B * L, D), M, Wvo, bvo)
    qm3 = qm.reshape(B, L, D)
    vp3 = vp.reshape(B, L, D)
    mv = _mean_corr(qm3, keys)
    idx, sw = _topk_weights_sc(mv, topk)
    return _aggregate(vp3, idx, sw, topk)


# speedup vs baseline: 1.6368x; 1.3831x over previous
import functools
import math

import jax
import jax.numpy as jnp
from jax import lax
from jax.experimental import pallas as pl
from jax.experimental.pallas import tpu as pltpu
from jax.experimental.pallas import tpu_sc as plsc


def _prep_body(wq, wk, wv, wo, bv, bo, m_ref, wvo_ref, bvo_ref):
    m_ref[...] = lax.dot_general(wq[...], wk[...], (((1,), (1,)), ((), ())),
                                 preferred_element_type=jnp.float32)
    wvo_ref[...] = jnp.dot(wv[...], wo[...],
                           preferred_element_type=jnp.float32).astype(jnp.bfloat16)
    bvo_ref[...] = jnp.dot(bv[...], wo[...],
                           preferred_element_type=jnp.float32) + bo[...]


def _prep(Wq, Wk, Wv, Wo, bv, bo):
    D = Wq.shape[0]
    fixed = lambda: (0, 0)
    return pl.pallas_call(
        _prep_body,
        in_specs=[pl.BlockSpec((D, D), fixed)] * 4
        + [pl.BlockSpec((1, D), fixed)] * 2,
        out_specs=[
            pl.BlockSpec((D, D), fixed),
            pl.BlockSpec((D, D), fixed),
            pl.BlockSpec((1, D), fixed),
        ],
        out_shape=[
            jax.ShapeDtypeStruct((D, D), jnp.float32),
            jax.ShapeDtypeStruct((D, D), jnp.bfloat16),
            jax.ShapeDtypeStruct((1, D), jnp.float32),
        ],
    )(Wq, Wk, Wv, Wo, bv.reshape(1, D), bo.reshape(1, D))


def _proj_body(xq, xv, m, wvo, bvo, qm_out, vp_out):
    qm_out[...] = jnp.dot(xq[...], m[...], preferred_element_type=jnp.float32)
    vp = jnp.dot(xv[...].astype(jnp.bfloat16), wvo[...],
                 preferred_element_type=jnp.float32) + bvo[...]
    vp_out[...] = vp.astype(jnp.bfloat16)


def _projections(xq, xv, M, Wvo, bvo):
    Mrows, D = xq.shape
    TM = 512
    row = lambda i: (i, 0)
    fixed = lambda i: (0, 0)
    return pl.pallas_call(
        _proj_body,
        grid=(Mrows // TM,),
        in_specs=[
            pl.BlockSpec((TM, D), row),
            pl.BlockSpec((TM, D), row),
            pl.BlockSpec((D, D), fixed),
            pl.BlockSpec((D, D), fixed),
            pl.BlockSpec((1, D), fixed),
        ],
        out_specs=[
            pl.BlockSpec((TM, D), row),
            pl.BlockSpec((TM, D), row),
        ],
        out_shape=[
            jax.ShapeDtypeStruct((Mrows, D), jnp.float32),
            jax.ShapeDtypeStruct((Mrows, D), jnp.bfloat16),
        ],
    )(xq, xv, M, Wvo, bvo)


def _corr_body(q_ref, k_ref, mv_ref):
    L = q_ref.shape[1]
    TJ = k_ref.shape[1]
    j = pl.program_id(1)
    q = q_ref[0]
    k = k_ref[0]
    g = lax.dot_general(q, k, (((1,), (1,)), ((), ())),
                        preferred_element_type=jnp.float32)
    lane = lax.broadcasted_iota(jnp.int32, (L, TJ), 1)
    x = g
    nbits = int(math.log2(TJ))
    for kb in range(3, nbits):
        sh = 1 << kb
        rolled = jnp.roll(x, -sh, axis=0)
        x = jnp.where((lane & sh) != 0, rolled, x)
    ci = lax.broadcasted_iota(jnp.int32, (TJ, 8), 0)
    ri = lax.broadcasted_iota(jnp.int32, (TJ, 8), 1)
    P = (ci % 8 == ri).astype(jnp.float32)
    s = jnp.dot(x, P, preferred_element_type=jnp.float32)
    lane8 = lax.broadcasted_iota(jnp.int32, (L, 8), 1)
    for kb in range(3):
        sh = 1 << kb
        s = jnp.where((lane8 & sh) != 0, jnp.roll(s, -sh, axis=0), s)
    contrib = jnp.sum(s, axis=1, keepdims=True)
    shifted = pltpu.roll(contrib, L - j * TJ, axis=0)
    scaled = shifted * (1.0 / q_ref.shape[2])

    @pl.when(j == 0)
    def _():
        mv_ref[0] = scaled

    @pl.when(j > 0)
    def _():
        mv_ref[0] += scaled


def _mean_corr(q3, k3):
    B, L, D = q3.shape
    TJ = 256
    grid = (B, L // TJ)
    mv = pl.pallas_call(
        _corr_body,
        grid=grid,
        in_specs=[
            pl.BlockSpec((1, L, D), lambda b, j: (b, 0, 0)),
            pl.BlockSpec((1, TJ, D), lambda b, j: (b, j, 0)),
        ],
        out_specs=pl.BlockSpec((1, L, 1), lambda b, j: (b, 0, 0)),
        out_shape=jax.ShapeDtypeStruct((B, L, 1), jnp.float32),
        compiler_params=pltpu.CompilerParams(
            dimension_semantics=("arbitrary", "arbitrary")),
    )(q3, k3)
    return mv.reshape(B, L)


_GDN = lax.GatherDimensionNumbers(
    offset_dims=(), collapsed_slice_dims=(0,), start_index_map=(0,))


def _lane_shuffle(x, idx):
    return lax.gather(x, idx[:, None], _GDN, slice_sizes=(1,),
                      mode=lax.GatherScatterMode.PROMISE_IN_BOUNDS)


def _all_lanes(x, op, lanes):
    for d in (1, 2, 4, 8):
        x = op(x, _lane_shuffle(x, lanes ^ d))
    return x


def _topk_sc_body(mv_hbm, idx_hbm, sw_hbm, mv_v, bm_v, idx_v, sw_v, *, topk):
    Bb, L = mv_hbm.shape
    c = lax.axis_index("c")
    s = lax.axis_index("s")

    @pl.when(jnp.logical_and(c == 0, s == 0))
    def _():
        pltpu.sync_copy(mv_hbm, mv_v)
        lanes = lax.iota(jnp.int32, 16)

        def bmloop(i, carry):
            acc = mv_v[0, pl.ds(i * 16, 16)]
            for b in range(1, Bb):
                acc = acc + mv_v[b, pl.ds(i * 16, 16)]
            bm_v[pl.ds(i * 16, 16)] = acc * (1.0 / Bb)
            return carry

        lax.fori_loop(0, L // 16, bmloop, 0)

        idx_acc = jnp.zeros((16,), jnp.int32)
        w_acc = [jnp.zeros((16,), jnp.float32) for _ in range(Bb)]
        for p in range(topk):
            def scanmax(j, carry):
                m_val, m_idx = carry
                v = bm_v[pl.ds(j * 16, 16)]
                idxs = j * 16 + lanes
                upd = v > m_val
                return (jnp.where(upd, v, m_val), jnp.where(upd, idxs, m_idx))

            m_val, m_idx = lax.fori_loop(
                0, L // 16, scanmax,
                (jnp.full((16,), -jnp.inf, jnp.float32),
                 jnp.zeros((16,), jnp.int32)))
            mxv = _all_lanes(m_val, jnp.maximum, lanes)
            cand = jnp.where(m_val == mxv, m_idx, jnp.int32(2**30))
            fidx = _all_lanes(cand, jnp.minimum, lanes)
            idx_acc = jnp.where(lanes == p, fidx, idx_acc)
            fidx_s = fidx[0]
            base = (fidx_s // 16) * 16
            hit = base + lanes == fidx_s
            ch = bm_v[pl.ds(base, 16)]
            bm_v[pl.ds(base, 16)] = jnp.where(
                hit, jnp.full((16,), -jnp.inf, jnp.float32), ch)
            for b in range(Bb):
                row = mv_v[b, pl.ds(base, 16)]
                wv = _all_lanes(jnp.where(hit, row, 0.0), jnp.add, lanes)
                w_acc[b] = jnp.where(lanes == p, wv, w_acc[b])

        idx_v[...] = idx_acc
        msk = lanes < topk
        for b in range(Bb):
            wm = jnp.where(msk, w_acc[b], -jnp.inf)
            mxw = _all_lanes(wm, jnp.maximum, lanes)
            e = jnp.where(msk, jnp.exp(wm - mxw), 0.0)
            ssum = _all_lanes(e, jnp.add, lanes)
            sw_v[b, pl.ds(0, 16)] = e / ssum
        pltpu.sync_copy(idx_v, idx_hbm)
        pltpu.sync_copy(sw_v, sw_hbm)


def _topk_weights_sc(mv, topk):
    B, L = mv.shape
    mesh = plsc.VectorSubcoreMesh(core_axis_name="c", subcore_axis_name="s")
    f = pl.kernel(
        functools.partial(_topk_sc_body, topk=topk),
        out_type=[
            jax.ShapeDtypeStruct((16,), jnp.int32),
            jax.ShapeDtypeStruct((B, 16), jnp.float32),
        ],
        mesh=mesh,
        scratch_types=[
            pltpu.VMEM((B, L), jnp.float32),
            pltpu.VMEM((L,), jnp.float32),
            pltpu.VMEM((16,), jnp.int32),
            pltpu.VMEM((B, 16), jnp.float32),
        ],
    )
    idx, sw = f(mv)
    return idx.reshape(1, 16), sw


def _agg_body(idx_ref, sw_ref, v_ref, out_ref, *, topk):
    b = pl.program_id(0)
    v = v_ref[0]
    L = v.shape[0]
    out_ref[0] = sw_ref[b, 0] * pltpu.roll(v, L - idx_ref[0, 0],
                                           axis=0).astype(jnp.float32)
    for i in range(1, topk):
        out_ref[0] += sw_ref[b, i] * pltpu.roll(v, L - idx_ref[0, i],
                                                axis=0).astype(jnp.float32)


def _aggregate(v3, idx, sw, topk):
    B, L, D = v3.shape
    out = pl.pallas_call(
        functools.partial(_agg_body, topk=topk),
        grid=(B,),
        in_specs=[
            pl.BlockSpec(memory_space=pltpu.SMEM),
            pl.BlockSpec(memory_space=pltpu.SMEM),
            pl.BlockSpec((1, L, D), lambda b: (b, 0, 0)),
        ],
        out_specs=pl.BlockSpec((1, L, D), lambda b: (b, 0, 0)),
        out_shape=jax.ShapeDtypeStruct((B, L, D), jnp.float32),
    )(idx, sw, v3)
    return out


def kernel(queries, keys, values, Wq, bq, Wk, bk, Wv, bv, Wo, bo):
    B, L, D = queries.shape
    topk = int(math.log(L))
    M, Wvo, bvo = _prep(Wq, Wk, Wv, Wo, bv, bo)
    qm, vp = _projections(
        queries.reshape(B * L, D), values.reshape(B * L, D), M, Wvo, bvo)
    qm3 = qm.reshape(B, L, D)
    vp3 = vp.reshape(B, L, D)
    mv = _mean_corr(qm3, keys)
    idx, sw = _topk_weights_sc(mv, topk)
    return _aggregate(vp3, idx, sw, topk)
